# baseline (speedup 1.0000x reference)
import functools

import jax
import jax.numpy as jnp
from jax import lax
from jax.experimental import pallas as pl
from jax.experimental.pallas import tpu as pltpu
from jax.experimental.pallas import tpu_sc as plsc

N_BLK = 256

_SC_NC = 2
_SC_NS = 16
_SC_CH = 128


def _seg_sum_sc_build(n_nodes, n_edges, h):
    mesh = plsc.VectorSubcoreMesh(core_axis_name="c", subcore_axis_name="s")
    n_chunks = n_edges // _SC_CH
    n_g = (n_chunks + _SC_NS - 1) // _SC_NS
    half = n_nodes // _SC_NC
    acc_rows = half // 8 * 8 + 128
    dump = acc_rows - 8
    stripe = (half // _SC_NS) // 8 * 8
    tail = half - _SC_NS * stripe
    zstripe = acc_rows // _SC_NS // 8 * 8
    ztail = acc_rows - _SC_NS * zstripe

    @functools.partial(
        pl.kernel, mesh=mesh,
        out_type=jax.ShapeDtypeStruct((n_nodes, h), jnp.float32),
        scratch_types=[
            pltpu.VMEM((_SC_CH,), jnp.int32),
            pltpu.VMEM((_SC_CH,), jnp.int32),
            pltpu.VMEM((_SC_CH,), jnp.int32),
            pltpu.VMEM((_SC_CH, 16), jnp.float32),
            pltpu.VMEM((_SC_CH, h), jnp.float32),
            pltpu.VMEM((16, h), jnp.float32),
            pltpu.VMEM_SHARED((acc_rows, h), jnp.float32),
            pltpu.SemaphoreType.DMA,
            pltpu.SemaphoreType.DMA,
        ],
    )
    def seg_sum(f_hbm, src_hbm, dst_hbm, w_hbm, out_hbm,
                src_v, dst_v, dsti_v, w_v, rows_v, zrow_v, acc_sh, sem, sem2):
        c = lax.axis_index("c")
        s = lax.axis_index("s")
        lo = c * half
        zv = jnp.zeros((16,), jnp.float32)
        for r16 in range(16):
            for j in range(h // 16):
                zrow_v[r16, pl.ds(j * 16, 16)] = zv

        def _zero(i, carry):
            pltpu.sync_copy(zrow_v, acc_sh.at[pl.ds(s * zstripe + i * 16, 16)])
            return carry

        lax.fori_loop(0, zstripe // 16, _zero, 0)

        @pl.when(s == _SC_NS - 1)
        def _():
            for t16 in range(ztail // 16):
                pltpu.sync_copy(
                    zrow_v,
                    acc_sh.at[pl.ds(_SC_NS * zstripe + t16 * 16, 16)])

        plsc.subcore_barrier()

        def _chunk(g, carry):
            cid = g * _SC_NS + s

            @pl.when(cid < n_chunks)
            def _():
                base = cid * _SC_CH
                a1 = pltpu.async_copy(src_hbm.at[pl.ds(base, _SC_CH)], src_v,
                                      sem)
                a2 = pltpu.async_copy(dst_hbm.at[pl.ds(base, _SC_CH)], dst_v,
                                      sem)
                a3 = pltpu.async_copy(w_hbm.at[pl.ds(base, _SC_CH)], w_v, sem)
                a1.wait()
                g_cp = pltpu.async_copy(f_hbm.at[src_v], rows_v, sem2)
                a2.wait()
                a3.wait()
                for b in range(_SC_CH // 16):
                    sl = pl.ds(b * 16, 16)
                    dl = dst_v[sl] - lo
                    inb = (dl >= 0) & (dl < half)
                    dsti_v[sl] = jnp.where(inb, dl, dump)
                g_cp.wait()

                def _wmul(e, wcarry):
                    wb = w_v[e]
                    for j in range(h // 16):
                        wsl = pl.ds(j * 16, 16)
                        rows_v[e, wsl] = rows_v[e, wsl] * wb
                    return wcarry

                lax.fori_loop(0, _SC_CH, _wmul, 0)
                pltpu.sync_copy(rows_v, acc_sh.at[dsti_v], add=True)

            return carry

        lax.fori_loop(0, n_g, _chunk, 0)
        plsc.subcore_barrier()
        pltpu.sync_copy(acc_sh.at[pl.ds(s * stripe, stripe)],
                        out_hbm.at[pl.ds(lo + s * stripe, stripe)])

        @pl.when(s == _SC_NS - 1)
        def _():
            pltpu.sync_copy(
                acc_sh.at[pl.ds(_SC_NS * stripe, tail)],
                out_hbm.at[pl.ds(lo + _SC_NS * stripe, tail)])

    return seg_sum


def _seg_sum_sc(f, src, dst, w):
    n_nodes, h = f.shape
    n_edges = src.shape[0]
    w16 = jnp.broadcast_to(w[:, None], (n_edges, 16))
    fn = _seg_sum_sc_build(n_nodes, n_edges, h)
    return fn(f, src, dst, w16)


def _re_gather_sc_build(n_rows, k, h):
    mesh = plsc.VectorSubcoreMesh(core_axis_name="c", subcore_axis_name="s")
    nw = _SC_NC * _SC_NS
    rpc = 8
    stripe = n_rows // nw // rpc * rpc
    n_ch = stripe // rpc
    n_tail = (n_rows - stripe * nw) // rpc

    @functools.partial(
        pl.kernel, mesh=mesh,
        out_type=jax.ShapeDtypeStruct((n_rows, h), jnp.float32),
        scratch_types=[
            pltpu.VMEM((rpc * k,), jnp.int32),
            pltpu.VMEM((rpc * k, 16), jnp.float32),
            pltpu.VMEM((rpc * k, h), jnp.float32),
            pltpu.VMEM((rpc, h), jnp.float32),
            pltpu.SemaphoreType.DMA,
            pltpu.SemaphoreType.DMA,
        ],
    )
    def re_gather(h_hbm, idx_hbm, w_hbm, out_hbm,
                  idx_v, w_v, rows_v, out_v, sem, sem2):
        c = lax.axis_index("c")
        s = lax.axis_index("s")
        w = s * _SC_NC + c

        def _do_chunk(row0):
            i0 = row0 * k
            a1 = pltpu.async_copy(idx_hbm.at[pl.ds(i0, rpc * k)], idx_v, sem)
            a1.wait()
            g_cp = pltpu.async_copy(h_hbm.at[idx_v], rows_v, sem2)
            a2 = pltpu.async_copy(w_hbm.at[pl.ds(i0, rpc * k)], w_v, sem)
            a2.wait()
            g_cp.wait()
            for r in range(rpc):
                for j in range(h // 16):
                    sl = pl.ds(j * 16, 16)
                    acc = rows_v[r * k, sl] * w_v[r * k]
                    for kk in range(1, k):
                        acc = acc + rows_v[r * k + kk, sl] * w_v[r * k + kk]
                    out_v[r, sl] = acc
            pltpu.sync_copy(out_v, out_hbm.at[pl.ds(row0, rpc)])

        def _chunk(g, carry):
            _do_chunk(w * stripe + g * rpc)
            return carry

        lax.fori_loop(0, n_ch, _chunk, 0)
        for t in range(n_tail):
            @pl.when(w == t)
            def _():
                _do_chunk(nw * stripe + t * rpc)

    return re_gather


def _re_gather_sc(hmat, idx, vals):
    n_rows, k = idx.shape
    h = hmat.shape[1]
    fn = _re_gather_sc_build(n_rows, k, h)
    idxf = idx.reshape(-1)
    wexp = jnp.broadcast_to(vals.reshape(-1)[:, None], (n_rows * k, 16))
    return fn(hmat, idxf, wexp)


def _prompt_body(f_ref, pt_ref, gt_ref, pre_ref, cw_ref, o_ref):
    f = f_ref[...]
    pt = pt_ref[...]
    x = jax.nn.relu(pt * f)
    x = gt_ref[...] * x
    x1 = pre_ref[...] * f
    hid = cw_ref[0, 0] * x + cw_ref[0, 1] * x1
    o_ref[...] = jnp.where(hid > 0, hid, jnp.exp(jnp.minimum(hid, 0.0)) - 1.0)


def _prompt_stage(features, pt, global_token, pre_token, combine_weight):
    n, h = features.shape
    grid = (pl.cdiv(n, N_BLK),)
    return pl.pallas_call(
        _prompt_body,
        grid=grid,
        in_specs=[
            pl.BlockSpec((N_BLK, h), lambda i: (i, 0)),
            pl.BlockSpec((1, h), lambda i: (0, 0)),
            pl.BlockSpec((1, h), lambda i: (0, 0)),
            pl.BlockSpec((1, h), lambda i: (0, 0)),
            pl.BlockSpec((1, 2), lambda i: (0, 0), memory_space=pltpu.SMEM),
        ],
        out_specs=pl.BlockSpec((N_BLK, h), lambda i: (i, 0)),
        out_shape=jax.ShapeDtypeStruct((n, h), jnp.float32),
    )(features, pt, global_token, pre_token, combine_weight)


def _znorm_body(f1_ref, agg_ref, bt_ref, z_ref):
    r = jnp.concatenate([f1_ref[...], agg_ref[...]], axis=1) * bt_ref[...]
    nrm = jnp.sqrt(jnp.sum(r * r, axis=1, keepdims=True))
    z_ref[...] = r / (nrm + 1e-8)


def _znorm_stage(features1, agg, balance_token, n_pad):
    n, h = features1.shape
    grid = (pl.cdiv(n_pad, N_BLK),)
    return pl.pallas_call(
        _znorm_body,
        grid=grid,
        in_specs=[
            pl.BlockSpec((N_BLK, h), lambda i: (i, 0)),
            pl.BlockSpec((N_BLK, h), lambda i: (i, 0)),
            pl.BlockSpec((1, 2 * h), lambda i: (0, 0)),
        ],
        out_specs=pl.BlockSpec((N_BLK, 2 * h), lambda i: (i, 0)),
        out_shape=jax.ShapeDtypeStruct((n_pad, 2 * h), jnp.float32),
    )(features1, agg, balance_token)


def _simtopk_body(n_valid_ref, zb_ref, zall_ref, vals_ref, idx_ref, cur_ref,
                  *, k, n_pad):
    rblk = zb_ref.shape[0]
    sim = jax.lax.dot_general(
        zb_ref[...], zall_ref[...], (((1,), (1,)), ((), ())),
        preferred_element_type=jnp.float32, precision=jax.lax.Precision.DEFAULT)
    ii = jax.lax.broadcasted_iota(jnp.int32, (rblk, n_pad), 1)
    n_valid = n_valid_ref[0]
    cur_ref[...] = jnp.where(ii < n_valid, sim, -jnp.inf)
    vals_l, idx_l = [], []
    v = cur_ref[...]
    m = jnp.max(v, axis=1)
    am = jnp.min(jnp.where(v == m[:, None], ii, n_pad), axis=1)
    vals_l.append(m)
    idx_l.append(am)
    for _ in range(k - 1):
        v = jnp.where(ii == am[:, None], -jnp.inf, cur_ref[...])
        cur_ref[...] = v
        m = jnp.max(v, axis=1)
        am = jnp.min(jnp.where(v == m[:, None], ii, n_pad), axis=1)
        vals_l.append(m)
        idx_l.append(am)
    vals = jnp.stack(vals_l, axis=1)
    idx = jnp.stack(idx_l, axis=1)
    vals = jax.nn.relu(vals)
    vals = vals / (jnp.sum(vals, axis=1, keepdims=True) + 1e-8)
    vals_ref[...] = vals
    idx_ref[...] = idx


def _simtopk_stage(z, n_valid, k=10, rblk=256):
    n_pad, h2 = z.shape
    grid = (n_pad // rblk,)
    nv = jnp.full((1,), n_valid, dtype=jnp.int32)
    return pl.pallas_call(
        functools.partial(_simtopk_body, k=k, n_pad=n_pad),
        grid=grid,
        in_specs=[
            pl.BlockSpec(memory_space=pltpu.SMEM),
            pl.BlockSpec((rblk, h2), lambda i: (i, 0)),
            pl.BlockSpec((n_pad, h2), lambda i: (0, 0)),
        ],
        out_specs=[
            pl.BlockSpec((rblk, k), lambda i: (i, 0)),
            pl.BlockSpec((rblk, k), lambda i: (i, 0)),
        ],
        out_shape=[
            jax.ShapeDtypeStruct((n_pad, k), jnp.float32),
            jax.ShapeDtypeStruct((n_pad, k), jnp.int32),
        ],
        scratch_shapes=[pltpu.VMEM((rblk, n_pad), jnp.float32)],
    )(nv, z, z)


def kernel(features, adj_indices, adj_values, down_k, tokens, wp_weight,
           global_token, pre_token, combine_weight, balance_token,
           W1, b1, W2, b2):
    n = features.shape[0]
    src = adj_indices[0]
    dst = adj_indices[1]
    pt = wp_weight @ tokens
    features1 = _prompt_stage(features, pt, global_token, pre_token,
                              combine_weight)

    agg = _seg_sum_sc(features1, src, dst, adj_values)
    n_pad = 10240
    f1_pad = jnp.pad(features1, ((0, n_pad - n), (0, 0)))
    agg_pad = jnp.pad(agg, ((0, n_pad - n), (0, 0)))
    z = _znorm_stage(f1_pad, agg_pad, balance_token, n_pad)
    vals, idx = _simtopk_stage(z, n)
    vals = vals[:n]
    idx = idx[:n]
    alpha = 0.5

    re1 = _re_gather_sc(features1, idx, vals)
    h1 = jax.nn.relu((alpha * agg + (1.0 - alpha) * re1) @ W1 + b1)
    h1W2 = h1 @ W2
    h1W2_p = jnp.pad(h1W2, ((0, 0), (0, 128 - h1W2.shape[1])))
    agg2 = _seg_sum_sc(h1W2_p, src, dst, adj_values)[:, :h1W2.shape[1]]
    re2 = _re_gather_sc(h1W2_p, idx, vals)[:, :h1W2.shape[1]]
    out = alpha * agg2 + (1.0 - alpha) * re2 + b2
    return out


# speedup vs baseline: 1.0723x; 1.0723x over previous
import functools

import jax
import jax.numpy as jnp
from jax import lax
from jax.experimental import pallas as pl
from jax.experimental.pallas import tpu as pltpu
from jax.experimental.pallas import tpu_sc as plsc

N_BLK = 256

_SC_NC = 2
_SC_NS = 16
_SC_CH = 128


def _seg_sum_sc_build(n_nodes, n_edges, h):
    mesh = plsc.VectorSubcoreMesh(core_axis_name="c", subcore_axis_name="s")
    n_chunks = n_edges // _SC_CH
    n_g = (n_chunks + _SC_NS - 1) // _SC_NS
    half = n_nodes // _SC_NC
    acc_rows = half // 8 * 8 + 128
    dump = acc_rows - 8
    stripe = (half // _SC_NS) // 8 * 8
    tail = half - _SC_NS * stripe
    zstripe = acc_rows // _SC_NS // 8 * 8
    ztail = acc_rows - _SC_NS * zstripe

    @functools.partial(
        pl.kernel, mesh=mesh,
        out_type=jax.ShapeDtypeStruct((n_nodes, h), jnp.float32),
        scratch_types=[
            pltpu.VMEM((_SC_CH,), jnp.int32),
            pltpu.VMEM((_SC_CH,), jnp.int32),
            pltpu.VMEM((_SC_CH,), jnp.int32),
            pltpu.VMEM((_SC_CH, 16), jnp.float32),
            pltpu.VMEM((_SC_CH, h), jnp.float32),
            pltpu.VMEM((16, h), jnp.float32),
            pltpu.VMEM_SHARED((acc_rows, h), jnp.float32),
            pltpu.SemaphoreType.DMA,
            pltpu.SemaphoreType.DMA,
        ],
    )
    def seg_sum(f_hbm, src_hbm, dst_hbm, w_hbm, out_hbm,
                src_v, dst_v, dsti_v, w_v, rows_v, zrow_v, acc_sh, sem, sem2):
        c = lax.axis_index("c")
        s = lax.axis_index("s")
        lo = c * half
        zv = jnp.zeros((16,), jnp.float32)
        for r16 in range(16):
            for j in range(h // 16):
                zrow_v[r16, pl.ds(j * 16, 16)] = zv

        def _zero(i, carry):
            pltpu.sync_copy(zrow_v, acc_sh.at[pl.ds(s * zstripe + i * 16, 16)])
            return carry

        lax.fori_loop(0, zstripe // 16, _zero, 0)

        @pl.when(s == _SC_NS - 1)
        def _():
            for t16 in range(ztail // 16):
                pltpu.sync_copy(
                    zrow_v,
                    acc_sh.at[pl.ds(_SC_NS * zstripe + t16 * 16, 16)])

        plsc.subcore_barrier()

        def _chunk(g, carry):
            cid = g * _SC_NS + s

            @pl.when(cid < n_chunks)
            def _():
                base = cid * _SC_CH
                a1 = pltpu.async_copy(src_hbm.at[pl.ds(base, _SC_CH)], src_v,
                                      sem)
                a2 = pltpu.async_copy(dst_hbm.at[pl.ds(base, _SC_CH)], dst_v,
                                      sem)
                a3 = pltpu.async_copy(w_hbm.at[pl.ds(base, _SC_CH)], w_v, sem)
                a1.wait()
                g_cp = pltpu.async_copy(f_hbm.at[src_v], rows_v, sem2)
                a2.wait()
                a3.wait()
                for b in range(_SC_CH // 16):
                    sl = pl.ds(b * 16, 16)
                    dl = dst_v[sl] - lo
                    inb = (dl >= 0) & (dl < half)
                    dsti_v[sl] = jnp.where(inb, dl, dump)
                g_cp.wait()

                def _wmul(e, wcarry):
                    wb = w_v[e]
                    for j in range(h // 16):
                        wsl = pl.ds(j * 16, 16)
                        rows_v[e, wsl] = rows_v[e, wsl] * wb
                    return wcarry

                lax.fori_loop(0, _SC_CH, _wmul, 0)
                pltpu.sync_copy(rows_v, acc_sh.at[dsti_v], add=True)

            return carry

        lax.fori_loop(0, n_g, _chunk, 0)
        plsc.subcore_barrier()
        pltpu.sync_copy(acc_sh.at[pl.ds(s * stripe, stripe)],
                        out_hbm.at[pl.ds(lo + s * stripe, stripe)])

        @pl.when(s == _SC_NS - 1)
        def _():
            pltpu.sync_copy(
                acc_sh.at[pl.ds(_SC_NS * stripe, tail)],
                out_hbm.at[pl.ds(lo + _SC_NS * stripe, tail)])

    return seg_sum


def _seg_sum_sc(f, src, dst, w):
    n_nodes, h = f.shape
    n_edges = src.shape[0]
    w16 = jnp.broadcast_to(w[:, None], (n_edges, 16))
    fn = _seg_sum_sc_build(n_nodes, n_edges, h)
    return fn(f, src, dst, w16)


def _prompt_body(f_ref, pt_ref, gt_ref, pre_ref, cw_ref, o_ref):
    f = f_ref[...]
    pt = pt_ref[...]
    x = jax.nn.relu(pt * f)
    x = gt_ref[...] * x
    x1 = pre_ref[...] * f
    hid = cw_ref[0, 0] * x + cw_ref[0, 1] * x1
    o_ref[...] = jnp.where(hid > 0, hid, jnp.exp(jnp.minimum(hid, 0.0)) - 1.0)


def _prompt_stage(features, pt, global_token, pre_token, combine_weight):
    n, h = features.shape
    grid = (pl.cdiv(n, N_BLK),)
    return pl.pallas_call(
        _prompt_body,
        grid=grid,
        in_specs=[
            pl.BlockSpec((N_BLK, h), lambda i: (i, 0)),
            pl.BlockSpec((1, h), lambda i: (0, 0)),
            pl.BlockSpec((1, h), lambda i: (0, 0)),
            pl.BlockSpec((1, h), lambda i: (0, 0)),
            pl.BlockSpec((1, 2), lambda i: (0, 0), memory_space=pltpu.SMEM),
        ],
        out_specs=pl.BlockSpec((N_BLK, h), lambda i: (i, 0)),
        out_shape=jax.ShapeDtypeStruct((n, h), jnp.float32),
    )(features, pt, global_token, pre_token, combine_weight)


def _znorm_body(f1_ref, agg_ref, bt_ref, z_ref):
    r = jnp.concatenate([f1_ref[...], agg_ref[...]], axis=1) * bt_ref[...]
    nrm = jnp.sqrt(jnp.sum(r * r, axis=1, keepdims=True))
    z_ref[...] = r / (nrm + 1e-8)


def _znorm_stage(features1, agg, balance_token, n_pad):
    n, h = features1.shape
    grid = (pl.cdiv(n_pad, N_BLK),)
    return pl.pallas_call(
        _znorm_body,
        grid=grid,
        in_specs=[
            pl.BlockSpec((N_BLK, h), lambda i: (i, 0)),
            pl.BlockSpec((N_BLK, h), lambda i: (i, 0)),
            pl.BlockSpec((1, 2 * h), lambda i: (0, 0)),
        ],
        out_specs=pl.BlockSpec((N_BLK, 2 * h), lambda i: (i, 0)),
        out_shape=jax.ShapeDtypeStruct((n_pad, 2 * h), jnp.float32),
    )(features1, agg, balance_token)


def _simtopk_body(n_valid_ref, zb_ref, zall_ref, vals_ref, idx_ref, cur_ref,
                  *, k, n_pad):
    rblk = zb_ref.shape[0]
    sim = jax.lax.dot_general(
        zb_ref[...], zall_ref[...], (((1,), (1,)), ((), ())),
        preferred_element_type=jnp.float32, precision=jax.lax.Precision.DEFAULT)
    ii = jax.lax.broadcasted_iota(jnp.int32, (rblk, n_pad), 1)
    n_valid = n_valid_ref[0]
    cur_ref[...] = jnp.where(ii < n_valid, sim, -jnp.inf)
    vals_l, idx_l = [], []
    v = cur_ref[...]
    m = jnp.max(v, axis=1)
    am = jnp.min(jnp.where(v == m[:, None], ii, n_pad), axis=1)
    vals_l.append(m)
    idx_l.append(am)
    for _ in range(k - 1):
        v = jnp.where(ii == am[:, None], -jnp.inf, cur_ref[...])
        cur_ref[...] = v
        m = jnp.max(v, axis=1)
        am = jnp.min(jnp.where(v == m[:, None], ii, n_pad), axis=1)
        vals_l.append(m)
        idx_l.append(am)
    vals = jnp.stack(vals_l, axis=1)
    idx = jnp.stack(idx_l, axis=1)
    vals = jax.nn.relu(vals)
    vals = vals / (jnp.sum(vals, axis=1, keepdims=True) + 1e-8)
    vals_ref[...] = vals
    idx_ref[...] = idx


def _simtopk_stage(z, n_valid, k=10, rblk=256):
    n_pad, h2 = z.shape
    grid = (n_pad // rblk,)
    nv = jnp.full((1,), n_valid, dtype=jnp.int32)
    return pl.pallas_call(
        functools.partial(_simtopk_body, k=k, n_pad=n_pad),
        grid=grid,
        in_specs=[
            pl.BlockSpec(memory_space=pltpu.SMEM),
            pl.BlockSpec((rblk, h2), lambda i: (i, 0)),
            pl.BlockSpec((n_pad, h2), lambda i: (0, 0)),
        ],
        out_specs=[
            pl.BlockSpec((rblk, k), lambda i: (i, 0)),
            pl.BlockSpec((rblk, k), lambda i: (i, 0)),
        ],
        out_shape=[
            jax.ShapeDtypeStruct((n_pad, k), jnp.float32),
            jax.ShapeDtypeStruct((n_pad, k), jnp.int32),
        ],
        scratch_shapes=[pltpu.VMEM((rblk, n_pad), jnp.float32)],
    )(nv, z, z)


def kernel(features, adj_indices, adj_values, down_k, tokens, wp_weight,
           global_token, pre_token, combine_weight, balance_token,
           W1, b1, W2, b2):
    n = features.shape[0]
    src = adj_indices[0]
    dst = adj_indices[1]
    pt = wp_weight @ tokens
    features1 = _prompt_stage(features, pt, global_token, pre_token,
                              combine_weight)

    agg = _seg_sum_sc(features1, src, dst, adj_values)
    n_pad = 10240
    f1_pad = jnp.pad(features1, ((0, n_pad - n), (0, 0)))
    agg_pad = jnp.pad(agg, ((0, n_pad - n), (0, 0)))
    z = _znorm_stage(f1_pad, agg_pad, balance_token, n_pad)
    vals, idx = _simtopk_stage(z, n)
    vals = vals[:n]
    idx = idx[:n]
    alpha = 0.5

    def spmm_re(h):
        return jnp.sum(vals[:, :, None] * h[idx], axis=1)

    h1 = jax.nn.relu((alpha * agg + (1.0 - alpha) * spmm_re(features1)) @ W1 + b1)
    h1W2 = h1 @ W2
    h1W2_p = jnp.pad(h1W2, ((0, 0), (0, 128 - h1W2.shape[1])))
    agg2 = _seg_sum_sc(h1W2_p, src, dst, adj_values)[:, :h1W2.shape[1]]
    out = alpha * agg2 + (1.0 - alpha) * spmm_re(h1W2) + b2
    return out
